# baseline (speedup 1.0000x reference)
import jax
import jax.numpy as jnp
from jax.experimental import pallas as pl
from jax.experimental.pallas import tpu as pltpu

_NCHUNKS = 4


def _dma_copy(x_ref, o_ref, *sems):
    rows = x_ref.shape[0]
    chunk = rows // _NCHUNKS
    copies = [
        pltpu.make_async_copy(
            x_ref.at[pl.ds(i * chunk, chunk)],
            o_ref.at[pl.ds(i * chunk, chunk)],
            sems[i],
        )
        for i in range(_NCHUNKS)
    ]
    for c in copies:
        c.start()
    for c in copies:
        c.wait()


def kernel(inputs_embeds):
    b, s, h = inputs_embeds.shape
    x = inputs_embeds.reshape(-1, 128)
    out = pl.pallas_call(
        _dma_copy,
        in_specs=[pl.BlockSpec(memory_space=pltpu.MemorySpace.HBM)],
        out_specs=pl.BlockSpec(memory_space=pltpu.MemorySpace.HBM),
        out_shape=jax.ShapeDtypeStruct(x.shape, x.dtype),
        scratch_shapes=[pltpu.SemaphoreType.DMA] * _NCHUNKS,
    )(x)
    return out.reshape(b, s, h)


# speedup vs baseline: 3.3953x; 3.3953x over previous
import jax
import jax.numpy as jnp
from jax.experimental import pallas as pl
from jax.experimental.pallas import tpu as pltpu


def _copy_block(x_ref, o_ref):
    o_ref[...] = x_ref[...]


def kernel(inputs_embeds):
    b, s, h = inputs_embeds.shape
    x = inputs_embeds.reshape(-1, 128)
    out = pl.pallas_call(
        _copy_block,
        out_shape=jax.ShapeDtypeStruct(x.shape, x.dtype),
    )(x)
    return out.reshape(b, s, h)


# speedup vs baseline: 4.8952x; 1.4417x over previous
import jax
import jax.numpy as jnp
from jax.experimental import pallas as pl
from jax.experimental.pallas import tpu as pltpu


def _copy_block(x_ref, o_ref):
    o_ref[...] = x_ref[...]


def kernel(inputs_embeds):
    return pl.pallas_call(
        _copy_block,
        out_shape=jax.ShapeDtypeStruct(inputs_embeds.shape, inputs_embeds.dtype),
    )(inputs_embeds)
